# baseline (speedup 1.0000x reference)
import functools

import jax
import jax.numpy as jnp
from jax.experimental import pallas as pl
from jax.experimental.pallas import tpu as pltpu

N = 10000
BI = 400
CW = 2048
NC = 5
CREM = N - (NC - 1) * CW
T1 = N // BI
BX = 2000


def _s1_body(x_ref, w1_ref, s1_ref):
    s1_ref[...] = jnp.dot(x_ref[...], w1_ref[...],
                          preferred_element_type=jnp.float32)


def _gcn_body(adja_ref, adjb_ref, s1_ref, b1_ref, w2_ref, b2_ref,
              out_ref, s2_ref, hacc_ref, oacc_ref):
    p = pl.program_id(0)
    i = pl.program_id(1)
    kc = pl.program_id(2)
    nclass = s2_ref.shape[1]
    row0 = i * BI

    @pl.when(p == 0)
    def _sweep1():
        def chunks(a, s_lo, s_hi):
            hacc_ref[...] = hacc_ref[...] + jnp.dot(
                a, s_lo, preferred_element_type=jnp.float32)
            oacc_ref[pl.ds(row0, BI), :] = oacc_ref[pl.ds(row0, BI), :] + jnp.dot(
                a, s_hi, preferred_element_type=jnp.float32)

        @pl.when(kc == 0)
        def _():
            hacc_ref[...] = jnp.zeros_like(hacc_ref)
            oacc_ref[pl.ds(row0, BI), :] = jnp.zeros((BI, nclass), jnp.float32)

        @pl.when(kc < NC - 1)
        def _():
            c0 = kc * CW
            a = adja_ref[:, pl.ds(c0, CW)]
            rows = c0 + jax.lax.broadcasted_iota(jnp.int32, (CW, nclass), 0)
            s2c = jnp.where(rows < row0, s2_ref[pl.ds(c0, CW), :], 0.0)
            chunks(a, s1_ref[pl.ds(c0, CW), :], s2c)

        @pl.when(kc == NC - 1)
        def _():
            c0 = (NC - 1) * CW
            a = adja_ref[:, c0:N]
            rows = c0 + jax.lax.broadcasted_iota(jnp.int32, (CREM, nclass), 0)
            s2c = jnp.where(rows < row0, s2_ref[pl.ds(c0, CREM), :], 0.0)
            chunks(a, s1_ref[pl.ds(c0, CREM), :], s2c)
            h = jnp.maximum(hacc_ref[...] + b1_ref[...], 0.0)
            s2_ref[pl.ds(row0, BI), :] = jnp.dot(
                h, w2_ref[...], preferred_element_type=jnp.float32)

    kc0 = (BI * i) // CW

    @pl.when((p == 1) & (kc >= kc0))
    def _sweep2():
        def suffix(a, c0, w):
            rows = c0 + jax.lax.broadcasted_iota(jnp.int32, (w, nclass), 0)
            s2c = jnp.where(rows >= row0, s2_ref[pl.ds(c0, w), :], 0.0)
            oacc_ref[pl.ds(row0, BI), :] = oacc_ref[pl.ds(row0, BI), :] + jnp.dot(
                a, s2c, preferred_element_type=jnp.float32)

        @pl.when(kc < NC - 1)
        def _():
            suffix(adjb_ref[...], kc * CW, CW)

        @pl.when(kc == NC - 1)
        def _():
            suffix(adjb_ref[:, :CREM], (NC - 1) * CW, CREM)

    @pl.when((p == 1) & (kc == NC - 1))
    def _emit():
        out_ref[...] = oacc_ref[pl.ds(row0, BI), :] + b2_ref[...]


@functools.partial(jax.jit, static_argnames=("interpret",))
def _gcn(x, adj, W1, b1, W2, b2, interpret=False):
    nfeat = x.shape[1]
    nhid = W1.shape[1]
    nclass = W2.shape[1]

    s1 = pl.pallas_call(
        _s1_body,
        grid=(N // BX,),
        in_specs=[
            pl.BlockSpec((BX, nfeat), lambda i: (i, 0)),
            pl.BlockSpec((nfeat, nhid), lambda i: (0, 0)),
        ],
        out_specs=pl.BlockSpec((BX, nhid), lambda i: (i, 0)),
        out_shape=jax.ShapeDtypeStruct((N, nhid), jnp.float32),
        interpret=interpret,
    )(x, W1)

    def adja_map(p, i, kc):
        return (jnp.where(p == 0, i, T1 - 1), 0)

    def adjb_map(p, i, kc):
        kc0 = (BI * i) // CW
        return (jnp.where(p == 0, 0, i), jnp.where(p == 0, 0, jnp.maximum(kc, kc0)))

    return pl.pallas_call(
        _gcn_body,
        grid=(2, T1, NC),
        in_specs=[
            pl.BlockSpec((BI, N), adja_map),
            pl.BlockSpec((BI, CW), adjb_map),
            pl.BlockSpec((N, nhid), lambda p, i, kc: (0, 0)),
            pl.BlockSpec((1, nhid), lambda p, i, kc: (0, 0)),
            pl.BlockSpec((nhid, nclass), lambda p, i, kc: (0, 0)),
            pl.BlockSpec((1, nclass), lambda p, i, kc: (0, 0)),
        ],
        out_specs=pl.BlockSpec(
            (BI, nclass), lambda p, i, kc: (jnp.where(p == 1, i, 0), 0)),
        out_shape=jax.ShapeDtypeStruct((N, nclass), jnp.float32),
        scratch_shapes=[
            pltpu.VMEM((N, nclass), jnp.float32),
            pltpu.VMEM((BI, nhid), jnp.float32),
            pltpu.VMEM((N, nclass), jnp.float32),
        ],
        interpret=interpret,
    )(adj, adj, s1, b1.reshape(1, -1), W2, b2.reshape(1, -1))


def kernel(x, adj, W1, b1, W2, b2):
    return _gcn(x, adj, W1, b1, W2, b2)


# speedup vs baseline: 1.5502x; 1.5502x over previous
import functools

import jax
import jax.numpy as jnp
from jax.experimental import pallas as pl
from jax.experimental.pallas import tpu as pltpu

N = 10000
BI = 400
BX = 2000


def _s1_body(x_ref, w1_ref, s1_ref):
    s1_ref[...] = jnp.dot(x_ref[...], w1_ref[...],
                          preferred_element_type=jnp.float32)


def _gcn_body(adj_ref, s1_ref, b1_ref, w2_ref, b2_ref, out_ref, s2_ref):
    p = pl.program_id(0)
    i = pl.program_id(1)

    @pl.when(p == 0)
    def _layer1():
        h = jnp.dot(adj_ref[...], s1_ref[...],
                    preferred_element_type=jnp.float32) + b1_ref[...]
        h = jnp.maximum(h, 0.0)
        s2_ref[pl.ds(i * BI, BI), :] = jnp.dot(
            h, w2_ref[...], preferred_element_type=jnp.float32)

    @pl.when(p == 1)
    def _layer2():
        out_ref[...] = jnp.dot(adj_ref[...], s2_ref[...],
                               preferred_element_type=jnp.float32) + b2_ref[...]


@functools.partial(jax.jit, static_argnames=("interpret",))
def _gcn(x, adj, W1, b1, W2, b2, interpret=False):
    nfeat = x.shape[1]
    nhid = W1.shape[1]
    nclass = W2.shape[1]

    s1 = pl.pallas_call(
        _s1_body,
        grid=(N // BX,),
        in_specs=[
            pl.BlockSpec((BX, nfeat), lambda i: (i, 0)),
            pl.BlockSpec((nfeat, nhid), lambda i: (0, 0)),
        ],
        out_specs=pl.BlockSpec((BX, nhid), lambda i: (i, 0)),
        out_shape=jax.ShapeDtypeStruct((N, nhid), jnp.float32),
        interpret=interpret,
    )(x, W1)

    return pl.pallas_call(
        _gcn_body,
        grid=(2, N // BI),
        in_specs=[
            pl.BlockSpec((BI, N), lambda p, i: (i, 0)),
            pl.BlockSpec((N, nhid), lambda p, i: (0, 0)),
            pl.BlockSpec((1, nhid), lambda p, i: (0, 0)),
            pl.BlockSpec((nhid, nclass), lambda p, i: (0, 0)),
            pl.BlockSpec((1, nclass), lambda p, i: (0, 0)),
        ],
        out_specs=pl.BlockSpec(
            (BI, nclass), lambda p, i: (jnp.where(p == 1, i, 0), 0)),
        out_shape=jax.ShapeDtypeStruct((N, nclass), jnp.float32),
        scratch_shapes=[
            pltpu.VMEM((N, nclass), jnp.float32),
        ],
        interpret=interpret,
    )(adj, s1, b1.reshape(1, -1), W2, b2.reshape(1, -1))


def kernel(x, adj, W1, b1, W2, b2):
    return _gcn(x, adj, W1, b1, W2, b2)


# speedup vs baseline: 1.5603x; 1.0065x over previous
import functools

import jax
import jax.numpy as jnp
from jax.experimental import pallas as pl
from jax.experimental.pallas import tpu as pltpu

N = 10000
BI = 400
BX = 2000


def _s1_body(x_ref, w1_ref, s1_ref):
    s1_ref[...] = jnp.dot(x_ref[...], w1_ref[...],
                          preferred_element_type=jnp.float32)


def _gcn_body(adj_ref, s1_ref, b1_ref, w2_ref, b2_ref, out_ref, s2_ref):
    p = pl.program_id(0)
    i = pl.program_id(1)

    @pl.when(p == 0)
    def _layer1():
        h = jnp.dot(adj_ref[...], s1_ref[...],
                    preferred_element_type=jnp.float32) + b1_ref[...]
        h = jnp.maximum(h, 0.0)
        s2_ref[pl.ds(i * BI, BI), :] = jnp.dot(
            h, w2_ref[...], preferred_element_type=jnp.float32)

    @pl.when(p == 1)
    def _layer2():
        out_ref[...] = jnp.dot(adj_ref[...], s2_ref[...],
                               preferred_element_type=jnp.float32) + b2_ref[...]


@functools.partial(jax.jit, static_argnames=("interpret",))
def _gcn(x, adj, W1, b1, W2, b2, interpret=False):
    nfeat = x.shape[1]
    nhid = W1.shape[1]
    nclass = W2.shape[1]

    s1 = pl.pallas_call(
        _s1_body,
        grid=(N // BX,),
        in_specs=[
            pl.BlockSpec((BX, nfeat), lambda i: (i, 0)),
            pl.BlockSpec((nfeat, nhid), lambda i: (0, 0)),
        ],
        out_specs=pl.BlockSpec((BX, nhid), lambda i: (i, 0)),
        out_shape=jax.ShapeDtypeStruct((N, nhid), jnp.float32),
        interpret=interpret,
    )(x, W1)

    return pl.pallas_call(
        _gcn_body,
        grid=(2, N // BI),
        in_specs=[
            pl.BlockSpec(
                (BI, N),
                lambda p, i: (jnp.where(p == 0, i, N // BI - 1 - i), 0)),
            pl.BlockSpec((N, nhid), lambda p, i: (0, 0)),
            pl.BlockSpec((1, nhid), lambda p, i: (0, 0)),
            pl.BlockSpec((nhid, nclass), lambda p, i: (0, 0)),
            pl.BlockSpec((1, nclass), lambda p, i: (0, 0)),
        ],
        out_specs=pl.BlockSpec(
            (BI, nclass),
            lambda p, i: (jnp.where(p == 1, N // BI - 1 - i, 0), 0)),
        out_shape=jax.ShapeDtypeStruct((N, nclass), jnp.float32),
        scratch_shapes=[
            pltpu.VMEM((N, nclass), jnp.float32),
        ],
        interpret=interpret,
    )(adj, s1, b1.reshape(1, -1), W2, b2.reshape(1, -1))


def kernel(x, adj, W1, b1, W2, b2):
    return _gcn(x, adj, W1, b1, W2, b2)


# speedup vs baseline: 1.6019x; 1.0267x over previous
import functools

import jax
import jax.numpy as jnp
from jax.experimental import pallas as pl
from jax.experimental.pallas import tpu as pltpu

N = 10000
BI = 400
BX = 2000


def _gcn_body(adj_ref, x_ref, w1_ref, b1_ref, w2_ref, b2_ref, out_ref,
              s1_ref, s2_ref):
    p = pl.program_id(0)
    i = pl.program_id(1)

    @pl.when((p == 0) & (i == 0))
    def _compute_s1():
        s1_ref[...] = jnp.dot(x_ref[...], w1_ref[...],
                              preferred_element_type=jnp.float32)

    @pl.when(p == 0)
    def _layer1():
        h = jnp.dot(adj_ref[...], s1_ref[...],
                    preferred_element_type=jnp.float32) + b1_ref[...]
        h = jnp.maximum(h, 0.0)
        s2_ref[pl.ds(i * BI, BI), :] = jnp.dot(
            h, w2_ref[...], preferred_element_type=jnp.float32)

    @pl.when(p == 1)
    def _layer2():
        out_ref[...] = jnp.dot(adj_ref[...], s2_ref[...],
                               preferred_element_type=jnp.float32) + b2_ref[...]


@functools.partial(jax.jit, static_argnames=("interpret",))
def _gcn(x, adj, W1, b1, W2, b2, interpret=False):
    nfeat = x.shape[1]
    nhid = W1.shape[1]
    nclass = W2.shape[1]

    return pl.pallas_call(
        _gcn_body,
        grid=(2, N // BI),
        in_specs=[
            pl.BlockSpec(
                (BI, N),
                lambda p, i: (jnp.where(p == 0, i, N // BI - 1 - i), 0)),
            pl.BlockSpec((N, nfeat), lambda p, i: (0, 0)),
            pl.BlockSpec((nfeat, nhid), lambda p, i: (0, 0)),
            pl.BlockSpec((1, nhid), lambda p, i: (0, 0)),
            pl.BlockSpec((nhid, nclass), lambda p, i: (0, 0)),
            pl.BlockSpec((1, nclass), lambda p, i: (0, 0)),
        ],
        out_specs=pl.BlockSpec(
            (BI, nclass),
            lambda p, i: (jnp.where(p == 1, N // BI - 1 - i, 0), 0)),
        out_shape=jax.ShapeDtypeStruct((N, nclass), jnp.float32),
        scratch_shapes=[
            pltpu.VMEM((N, nhid), jnp.float32),
            pltpu.VMEM((N, nclass), jnp.float32),
        ],
        interpret=interpret,
    )(adj, x, W1, b1.reshape(1, -1), W2, b2.reshape(1, -1))


def kernel(x, adj, W1, b1, W2, b2):
    return _gcn(x, adj, W1, b1, W2, b2)
